# baseline (speedup 1.0000x reference)
import jax
import jax.numpy as jnp
from jax import lax
from jax.experimental import pallas as pl
from jax.experimental.pallas import tpu as pltpu
from jax.experimental.pallas import tpu_sc as plsc

_NC, _NS = 2, 16
_NW = _NC * _NS
_LG = 8
_BB = 32


def _make_body(B, L, D):
    nlg = L // _LG
    nbb = B // _BB
    ntasks = nlg * nbb
    tpw = ntasks // _NW

    def body(xt_hbm, tbl_hbm, out_hbm, ib0, ib1, ob0, ob1, tblv,
             sin0, sin1, sout0, sout1):
        ibufs = (ib0, ib1)
        obufs = (ob0, ob1)
        sins = (sin0, sin1)
        souts = (sout0, sout1)
        wid = lax.axis_index("s") * _NC + lax.axis_index("c")
        base = wid * tpw

        def decode(t):
            lg = t // nbb
            bb = t % nbb
            return pl.multiple_of(lg * _LG, _LG), bb * _BB

        def in_copy(n, t):
            l0, b0 = decode(t)
            return pltpu.make_async_copy(
                xt_hbm.at[pl.ds(l0, _LG), :, pl.ds(b0, _BB)], ibufs[n], sins[n])

        def out_copy(n, t):
            l0, b0 = decode(t)
            return pltpu.make_async_copy(
                obufs[n], out_hbm.at[pl.ds(b0, _BB), pl.ds(l0, _LG), :],
                souts[n])

        iota = lax.iota(jnp.int32, 16)
        idx_d = [iota + 16 * k for k in range(4)]
        idx_l = [jnp.full((16,), lrel, jnp.int32) for lrel in range(_LG)]

        def fill_table_half(n, t):
            l0, _ = decode(t)
            rows = [[tblv[l0 + lrel, pl.ds(16 * k, 16)] for k in range(4)]
                    for lrel in range(_LG)]

            def frow(brel, carry):
                for lrel in range(_LG):
                    for k in range(4):
                        obufs[n][brel, lrel, pl.ds(D + 16 * k, 16)] = \
                            rows[lrel][k]
                return carry

            lax.fori_loop(0, _BB, frow, 0)

        def build(n, t):
            def brow(brel, carry):
                ib = jnp.full((16,), brel, jnp.int32)
                for lrel in range(_LG):
                    for k in range(4):
                        v = plsc.load_gather(
                            ibufs[n], [idx_l[lrel], idx_d[k], ib])
                        obufs[n][brel, lrel, pl.ds(16 * k, 16)] = v
                return carry

            lax.fori_loop(0, _BB, brow, 0)

        pltpu.sync_copy(tbl_hbm, tblv)
        for n in range(2):
            in_copy(n, base + n).start()
        for n in range(2):
            in_copy(n, base + n).wait()
            fill_table_half(n, base + n)
            build(n, base + n)
            in_copy(n, base + n + 2).start()
            out_copy(n, base + n).start()

        def chunk(c, carry):
            k0 = base + 2 * c
            for n in range(2):
                k = k0 + n
                out_copy(n, k - 2).wait()
                in_copy(n, k).wait()

                lg_prev = (k - 2) // nbb
                lg_cur = k // nbb

                @pl.when(lg_prev != lg_cur)
                def _():
                    fill_table_half(n, k)

                build(n, k)

                @pl.when(k + 2 < base + tpw)
                def _():
                    in_copy(n, k + 2).start()

                out_copy(n, k).start()
            return carry

        lax.fori_loop(1, tpw // 2, chunk, 0)
        for n in range(2):
            out_copy(n, base + tpw - 2 + n).wait()

    return body


def kernel(x, position_table):
    B, L, D = x.shape
    xt = jnp.transpose(x, (1, 2, 0))
    tbl = position_table[:L]
    mesh = plsc.VectorSubcoreMesh(core_axis_name="c", subcore_axis_name="s")
    f = pl.kernel(
        _make_body(B, L, D),
        out_type=jax.ShapeDtypeStruct((B, L, 2 * D), x.dtype),
        mesh=mesh,
        scratch_types=(
            [pltpu.VMEM((_LG, D, _BB), x.dtype) for _ in range(2)]
            + [pltpu.VMEM((_BB, _LG, 2 * D), x.dtype) for _ in range(2)]
            + [pltpu.VMEM((L, D), x.dtype)]
            + [pltpu.SemaphoreType.DMA for _ in range(4)]
        ),
        compiler_params=pltpu.CompilerParams(
            use_tc_tiling_on_sc=False, needs_layout_passes=False),
    )
    return f(xt, tbl)


# speedup vs baseline: 2.1550x; 2.1550x over previous
import jax
import jax.numpy as jnp
from jax import lax
from jax.experimental import pallas as pl
from jax.experimental.pallas import tpu as pltpu
from jax.experimental.pallas import tpu_sc as plsc

_NC, _NS = 2, 16
_NW = _NC * _NS
_NCHUNK = 4


def _make_body(L, D, bpw, slab_base, writes_out):
    def body(xc_hbm, tbl_hbm, out_hbm, xv0, xv1, buf0, buf1,
             sin0, sin1, sout0, sout1):
        xvs = (xv0, xv1)
        bufs = (buf0, buf1)
        sins = (sin0, sin1)
        souts = (sout0, sout1)
        wid = lax.axis_index("s") * _NC + lax.axis_index("c")
        base = wid * bpw

        def in_copy(n, j):
            return pltpu.make_async_copy(xc_hbm.at[base + j], xvs[n], sins[n])

        def out_copy(n, j):
            return pltpu.make_async_copy(
                bufs[n], out_hbm.at[slab_base + base + j], souts[n])

        def assemble(n):
            def rowpair(j, carry):
                for k in range(4):
                    bufs[n][2 * j, pl.ds(16 * k, 16)] = \
                        xvs[n][j, pl.ds(16 * k, 16)]
                for k in range(4):
                    bufs[n][2 * j + 1, pl.ds(16 * k, 16)] = \
                        xvs[n][j, pl.ds(D + 16 * k, 16)]
                return carry
            lax.fori_loop(0, L // 2, rowpair, 0)

        for n in range(2):
            pltpu.sync_copy(tbl_hbm, bufs[n])
        for n in range(2):
            in_copy(n, n).start()
        for n in range(2):
            in_copy(n, n).wait()
            assemble(n)
            out_copy(n, n).start()
            in_copy(n, n + 2).start()

        def chunk(c, carry):
            j0 = c * 2
            for n in range(2):
                in_copy(n, j0 + n).wait()
                out_copy(n, j0 + n - 2).wait()
                assemble(n)
                out_copy(n, j0 + n).start()

                @pl.when(j0 + n + 2 < bpw)
                def _():
                    in_copy(n, j0 + n + 2).start()
            return carry

        lax.fori_loop(1, bpw // 2, chunk, 0)
        for n in range(2):
            out_copy(n, bpw - 2 + n).wait()

    return body


def kernel(x, position_table):
    B, L, D = x.shape
    bc = B // _NCHUNK
    bpw = bc // _NW
    tbl = jnp.concatenate(
        [jnp.zeros((L, D), x.dtype), position_table[:L]], axis=-1)
    chunks = [
        lax.slice_in_dim(x, i * bc, (i + 1) * bc, axis=0)
        .reshape(bc, L // 2, 2 * D)
        for i in range(_NCHUNK)
    ]
    mesh = plsc.VectorSubcoreMesh(core_axis_name="c", subcore_axis_name="s")
    scratch = (
        [pltpu.VMEM((L // 2, 2 * D), x.dtype) for _ in range(2)]
        + [pltpu.VMEM((L, 2 * D), x.dtype) for _ in range(2)]
        + [pltpu.SemaphoreType.DMA for _ in range(4)]
    )
    params = pltpu.CompilerParams(use_tc_tiling_on_sc=True)

    first = pl.kernel(
        _make_body(L, D, bpw, 0, True),
        out_type=jax.ShapeDtypeStruct((B, L, 2 * D), x.dtype),
        mesh=mesh,
        scratch_types=scratch,
        compiler_params=params,
    )
    out_ref = jax.new_ref(first(chunks[0], tbl))
    for i in range(1, _NCHUNK):
        rest = pl.kernel(
            _make_body(L, D, bpw, i * bc, False),
            out_type=(),
            mesh=mesh,
            scratch_types=scratch,
            compiler_params=params,
        )
        rest(chunks[i], tbl, out_ref)
    return out_ref[...]


# speedup vs baseline: 2.3949x; 1.1113x over previous
import jax
import jax.numpy as jnp
from jax import lax
from jax.experimental import pallas as pl
from jax.experimental.pallas import tpu as pltpu
from jax.experimental.pallas import tpu_sc as plsc

_NC, _NS = 2, 16
_NW = _NC * _NS
_NCHUNK = 1


def _make_body(L, D, bpw, slab_base, writes_out):
    def body(xc_hbm, tbl_hbm, out_hbm, xv0, xv1, buf0, buf1,
             sin0, sin1, sout0, sout1):
        xvs = (xv0, xv1)
        bufs = (buf0, buf1)
        sins = (sin0, sin1)
        souts = (sout0, sout1)
        wid = lax.axis_index("s") * _NC + lax.axis_index("c")
        base = wid * bpw

        def in_copy(n, j):
            return pltpu.make_async_copy(xc_hbm.at[base + j], xvs[n], sins[n])

        def out_copy(n, j):
            return pltpu.make_async_copy(
                bufs[n], out_hbm.at[slab_base + base + j], souts[n])

        def assemble(n):
            def rowpair(j, carry):
                for k in range(4):
                    bufs[n][2 * j, pl.ds(16 * k, 16)] = \
                        xvs[n][j, pl.ds(16 * k, 16)]
                for k in range(4):
                    bufs[n][2 * j + 1, pl.ds(16 * k, 16)] = \
                        xvs[n][j, pl.ds(D + 16 * k, 16)]
                return carry
            lax.fori_loop(0, L // 2, rowpair, 0)

        for n in range(2):
            pltpu.sync_copy(tbl_hbm, bufs[n])
        for n in range(2):
            in_copy(n, n).start()
        for n in range(2):
            in_copy(n, n).wait()
            assemble(n)
            out_copy(n, n).start()
            in_copy(n, n + 2).start()

        def chunk(c, carry):
            j0 = c * 2
            for n in range(2):
                in_copy(n, j0 + n).wait()
                out_copy(n, j0 + n - 2).wait()
                assemble(n)
                out_copy(n, j0 + n).start()

                @pl.when(j0 + n + 2 < bpw)
                def _():
                    in_copy(n, j0 + n + 2).start()
            return carry

        lax.fori_loop(1, bpw // 2, chunk, 0)
        for n in range(2):
            out_copy(n, bpw - 2 + n).wait()

    return body


def kernel(x, position_table):
    B, L, D = x.shape
    bc = B // _NCHUNK
    bpw = bc // _NW
    tbl = jnp.concatenate(
        [jnp.zeros((L, D), x.dtype), position_table[:L]], axis=-1)
    chunks = [
        lax.slice_in_dim(x, i * bc, (i + 1) * bc, axis=0)
        .reshape(bc, L // 2, 2 * D)
        for i in range(_NCHUNK)
    ] if _NCHUNK > 1 else [x.reshape(B, L // 2, 2 * D)]
    mesh = plsc.VectorSubcoreMesh(core_axis_name="c", subcore_axis_name="s")
    scratch = (
        [pltpu.VMEM((L // 2, 2 * D), x.dtype) for _ in range(2)]
        + [pltpu.VMEM((L, 2 * D), x.dtype) for _ in range(2)]
        + [pltpu.SemaphoreType.DMA for _ in range(4)]
    )
    params = pltpu.CompilerParams(use_tc_tiling_on_sc=True)

    first = pl.kernel(
        _make_body(L, D, bpw, 0, True),
        out_type=jax.ShapeDtypeStruct((B, L, 2 * D), x.dtype),
        mesh=mesh,
        scratch_types=scratch,
        compiler_params=params,
    )
    if _NCHUNK == 1:
        return first(chunks[0], tbl)
    out_ref = jax.new_ref(first(chunks[0], tbl))
    for i in range(1, _NCHUNK):
        rest = pl.kernel(
            _make_body(L, D, bpw, i * bc, False),
            out_type=(),
            mesh=mesh,
            scratch_types=scratch,
            compiler_params=params,
        )
        rest(chunks[i], tbl, out_ref)
    return out_ref[...]
